# baseline (speedup 1.0000x reference)
import jax
import jax.numpy as jnp
from jax import lax
from jax.experimental import pallas as pl
from jax.experimental.pallas import tpu as pltpu

S, B, D, H, A = 2048, 16, 64, 128, 16
HALF = S // 2
B2 = 2 * B
T_BLK = 256
N_BLK = HALF // T_BLK


def _gru_kernel(x_ref, draw_ref, ks_ref, ksn_ref, xB_ref, h0_ref,
                Wi_ref, Wh_ref, bfold_ref, bhn_ref, Wout_ref, bout_ref,
                out_ref, h_ref, gi_ref, hs_ref):
    i = pl.program_id(0)

    @pl.when(i == 0)
    def _():
        d0 = draw_ref[0, :B].astype(jnp.float32)[:, None]
        hA = h0_ref[0] * (1.0 - d0)
        h_ref[...] = jnp.concatenate(
            [hA, jnp.zeros((B, H), jnp.float32)], axis=0)

    x = x_ref[...].reshape(T_BLK * B2, D)
    gi = jnp.dot(x, Wi_ref[...], preferred_element_type=jnp.float32)
    gi_ref[...] = (gi + bfold_ref[0]).reshape(T_BLK, B2, 3 * H)

    Wh = Wh_ref[...]
    bhn = bhn_ref[0]

    def step(t, h):
        m = jnp.dot(h, Wh, preferred_element_type=jnp.float32)
        k = 1.0 - draw_ref[i * T_BLK + t, :].astype(jnp.float32)[:, None]
        gh = m * k
        h_m = h * k
        gi_t = gi_ref[t]
        r = jax.nn.sigmoid(gi_t[:, :H] + gh[:, :H])
        z = jax.nn.sigmoid(gi_t[:, H:2 * H] + gh[:, H:2 * H])
        n = jnp.tanh(gi_t[:, 2 * H:] + r * (gh[:, 2 * H:] + bhn))
        h_new = n + z * (h_m - n)
        hs_ref[t] = h_new
        return h_new

    h_final = lax.fori_loop(0, T_BLK, step, h_ref[...], unroll=16)
    h_ref[...] = h_final

    hs = hs_ref[...].reshape(T_BLK * B2, H)
    o = jnp.dot(hs, Wout_ref[...], preferred_element_type=jnp.float32)
    o = (o + bout_ref[0]).reshape(T_BLK, B2, A)
    out_ref[pl.ds(i * T_BLK, T_BLK)] = o

    @pl.when(i == N_BLK - 1)
    def _():
        db = draw_ref[:, B:]
        tv = lax.broadcasted_iota(jnp.int32, (HALF, B), 0)
        first = jnp.min(jnp.where(db > 0, tv, HALF), axis=0)
        L = jnp.max(first)

        Wi = Wi_ref[...]
        Wout = Wout_ref[...]
        bout = bout_ref[0]
        bfold = bfold_ref[0]

        def fstep(t, hf):
            m = jnp.dot(hf, Wh, preferred_element_type=jnp.float32)
            gx = jnp.dot(xB_ref[t], Wi,
                         preferred_element_type=jnp.float32) + bfold
            k = 1.0 - draw_ref[t, B:].astype(jnp.float32)[:, None]
            gh = m * k
            h_m = hf * k
            r = jax.nn.sigmoid(gx[:, :H] + gh[:, :H])
            z = jax.nn.sigmoid(gx[:, H:2 * H] + gh[:, H:2 * H])
            n = jnp.tanh(gx[:, 2 * H:] + r * (gh[:, 2 * H:] + bhn))
            h_new = n + z * (h_m - n)
            out_ref[t, B:] = jnp.dot(
                h_new, Wout, preferred_element_type=jnp.float32) + bout
            return h_new

        lax.fori_loop(0, L, fstep, h_ref[:B, :])

        keep = (1 - ks_ref[...] * ksn_ref[...]).astype(jnp.float32)
        out_ref[...] = out_ref[...] * keep[:, :, None]


def _fold(a):
    return jnp.swapaxes(a.reshape((2, HALF) + a.shape[1:]), 0, 1).reshape(
        (HALF, 2 * a.shape[1]) + a.shape[2:])


@jax.jit
def kernel(obs, hidden_states, dones, Wi, Wh, bi, bh, Wout, bout):
    x2 = obs.reshape(S, B, D)
    d2 = dones.reshape(S, B)
    split = d2.at[0, :].set(1)
    split_next = jnp.concatenate(
        [split[1:], jnp.ones((1, B), dtype=split.dtype)], axis=0)
    bfold = bi + jnp.concatenate([bh[:2 * H], jnp.zeros((H,), bh.dtype)])
    bhn = bh[2 * H:]

    full = lambda s: pl.BlockSpec(s, lambda i: tuple(0 for _ in s))
    out = pl.pallas_call(
        _gru_kernel,
        grid=(N_BLK,),
        in_specs=[
            pl.BlockSpec((T_BLK, B2, D), lambda i: (i, 0, 0)),
            full((HALF, B2)),
            full((HALF, B2)),
            full((HALF, B2)),
            full((HALF, B, D)),
            full((1, B, H)),
            full((D, 3 * H)),
            full((H, 3 * H)),
            full((1, 3 * H)),
            full((1, H)),
            full((H, A)),
            full((1, A)),
        ],
        out_specs=pl.BlockSpec((HALF, B2, A), lambda i: (0, 0, 0)),
        out_shape=jax.ShapeDtypeStruct((HALF, B2, A), jnp.float32),
        scratch_shapes=[
            pltpu.VMEM((B2, H), jnp.float32),
            pltpu.VMEM((T_BLK, B2, 3 * H), jnp.float32),
            pltpu.VMEM((T_BLK, B2, H), jnp.float32),
        ],
    )(_fold(x2), _fold(d2), _fold(split), _fold(split_next),
      x2[HALF:], hidden_states,
      Wi, Wh, bfold.reshape(1, 3 * H), bhn.reshape(1, H),
      Wout, bout.reshape(1, A))
    return jnp.swapaxes(out.reshape(HALF, 2, B, A), 0, 1).reshape(S * B, A)


# speedup vs baseline: 1.1407x; 1.1407x over previous
import jax
import jax.numpy as jnp
from jax import lax
from jax.experimental import pallas as pl
from jax.experimental.pallas import tpu as pltpu

S, B, D, H, A = 2048, 16, 64, 128, 16
HALF = S // 2
B2 = 2 * B
T_BLK = 128
N_BLK = HALF // T_BLK


def _gru_kernel(x_ref, draw_ref, ks_ref, ksn_ref, xB_ref, h0_ref,
                Wi_ref, Wh_ref, bfold_ref, bhn_ref, Wout_ref, bout_ref,
                out_ref, h_ref, gi_ref, hs_ref, hfix_ref):
    i = pl.program_id(0)

    @pl.when(i == 0)
    def _():
        d0 = draw_ref[0, :B].astype(jnp.float32)[:, None]
        hA = h0_ref[0] * (1.0 - d0)
        h_ref[...] = jnp.concatenate(
            [hA, jnp.zeros((B, H), jnp.float32)], axis=0)

    x = x_ref[...].reshape(T_BLK * B2, D)
    gi = jnp.dot(x, Wi_ref[...], preferred_element_type=jnp.float32)
    gi_ref[...] = (gi + bfold_ref[0]).reshape(T_BLK, B2, 3 * H)

    Wh = Wh_ref[...]
    bhn = bhn_ref[0]

    def step(t, h):
        m = jnp.dot(h, Wh, preferred_element_type=jnp.float32)
        k = 1.0 - draw_ref[i * T_BLK + t, :].astype(jnp.float32)[:, None]
        gh = m * k
        h_m = h * k
        gi_t = gi_ref[t]
        r = jax.nn.sigmoid(gi_t[:, :H] + gh[:, :H])
        z = jax.nn.sigmoid(gi_t[:, H:2 * H] + gh[:, H:2 * H])
        n = jnp.tanh(gi_t[:, 2 * H:] + r * (gh[:, 2 * H:] + bhn))
        h_new = n + z * (h_m - n)
        hs_ref[t] = h_new
        return h_new

    h_final = lax.fori_loop(0, T_BLK, step, h_ref[...], unroll=16)
    h_ref[...] = h_final

    hs = hs_ref[...].reshape(T_BLK * B2, H)
    o = jnp.dot(hs, Wout_ref[...], preferred_element_type=jnp.float32)
    o = (o + bout_ref[0]).reshape(T_BLK, B2, A)
    out_ref[pl.ds(i * T_BLK, T_BLK)] = o

    @pl.when(i == N_BLK - 1)
    def _():
        db = draw_ref[:, B:]
        tv = lax.broadcasted_iota(jnp.int32, (HALF, B), 0)
        first = jnp.min(jnp.where(db > 0, tv, HALF), axis=0)
        L = jnp.max(first)

        Wi = Wi_ref[...]
        Wout = Wout_ref[...]
        bout = bout_ref[0]
        bfold = bfold_ref[0]

        def fstep(t, hf):
            m = jnp.dot(hf, Wh, preferred_element_type=jnp.float32)
            gx = jnp.dot(xB_ref[t], Wi,
                         preferred_element_type=jnp.float32) + bfold
            k = 1.0 - draw_ref[t, B:].astype(jnp.float32)[:, None]
            gh = m * k
            h_m = hf * k
            r = jax.nn.sigmoid(gx[:, :H] + gh[:, :H])
            z = jax.nn.sigmoid(gx[:, H:2 * H] + gh[:, H:2 * H])
            n = jnp.tanh(gx[:, 2 * H:] + r * (gh[:, 2 * H:] + bhn))
            h_new = n + z * (h_m - n)
            hfix_ref[t] = h_new
            return h_new

        lax.fori_loop(0, L, fstep, h_ref[:B, :])

        of = jnp.dot(hfix_ref[...].reshape(HALF * B, H), Wout,
                     preferred_element_type=jnp.float32) + bout
        of = of.reshape(HALF, B, A)
        sel = (lax.broadcasted_iota(jnp.int32, (HALF, B, A), 0) < L)
        out_ref[:, B:] = jnp.where(sel, of, out_ref[:, B:])

        keep = (1 - ks_ref[...] * ksn_ref[...]).astype(jnp.float32)
        out_ref[...] = out_ref[...] * keep[:, :, None]


def _fold(a):
    return jnp.swapaxes(a.reshape((2, HALF) + a.shape[1:]), 0, 1).reshape(
        (HALF, 2 * a.shape[1]) + a.shape[2:])


@jax.jit
def kernel(obs, hidden_states, dones, Wi, Wh, bi, bh, Wout, bout):
    x2 = obs.reshape(S, B, D)
    d2 = dones.reshape(S, B)
    split = d2.at[0, :].set(1)
    split_next = jnp.concatenate(
        [split[1:], jnp.ones((1, B), dtype=split.dtype)], axis=0)
    bfold = bi + jnp.concatenate([bh[:2 * H], jnp.zeros((H,), bh.dtype)])
    bhn = bh[2 * H:]

    full = lambda s: pl.BlockSpec(s, lambda i: tuple(0 for _ in s))
    out = pl.pallas_call(
        _gru_kernel,
        grid=(N_BLK,),
        in_specs=[
            pl.BlockSpec((T_BLK, B2, D), lambda i: (i, 0, 0)),
            full((HALF, B2)),
            full((HALF, B2)),
            full((HALF, B2)),
            full((HALF, B, D)),
            full((1, B, H)),
            full((D, 3 * H)),
            full((H, 3 * H)),
            full((1, 3 * H)),
            full((1, H)),
            full((H, A)),
            full((1, A)),
        ],
        out_specs=pl.BlockSpec((HALF, B2, A), lambda i: (0, 0, 0)),
        out_shape=jax.ShapeDtypeStruct((HALF, B2, A), jnp.float32),
        scratch_shapes=[
            pltpu.VMEM((B2, H), jnp.float32),
            pltpu.VMEM((T_BLK, B2, 3 * H), jnp.float32),
            pltpu.VMEM((T_BLK, B2, H), jnp.float32),
            pltpu.VMEM((HALF, B, H), jnp.float32),
        ],
    )(_fold(x2), _fold(d2), _fold(split), _fold(split_next),
      x2[HALF:], hidden_states,
      Wi, Wh, bfold.reshape(1, 3 * H), bhn.reshape(1, H),
      Wout, bout.reshape(1, A))
    return jnp.swapaxes(out.reshape(HALF, 2, B, A), 0, 1).reshape(S * B, A)


# speedup vs baseline: 1.1665x; 1.0226x over previous
import jax
import jax.numpy as jnp
from jax import lax
from jax.experimental import pallas as pl
from jax.experimental.pallas import tpu as pltpu

S, B, D, H, A = 2048, 16, 64, 128, 16
HALF = S // 2
B2 = 2 * B
T_BLK = 128
N_BLK = HALF // T_BLK


def _gru_kernel(x_ref, draw_ref, ks_ref, ksn_ref, xB_ref, h0_ref,
                Wi_ref, Wh_ref, bfold_ref, bhn_ref, Wout_ref, bout_ref,
                out_ref, h_ref, gi_ref, hs_ref, hfix_ref):
    i = pl.program_id(0)

    @pl.when(i == 0)
    def _():
        d0 = draw_ref[0, :B].astype(jnp.float32)[:, None]
        hA = h0_ref[0] * (1.0 - d0)
        h_ref[...] = jnp.concatenate(
            [hA, jnp.zeros((B, H), jnp.float32)], axis=0)

    x = x_ref[...].reshape(T_BLK * B2, D)
    gi = jnp.dot(x, Wi_ref[...], preferred_element_type=jnp.float32)
    gi_ref[...] = (gi + bfold_ref[0]).reshape(T_BLK, B2, 3 * H)

    Wh = Wh_ref[...]
    bhn = bhn_ref[0]

    def step(t, h):
        m = jnp.dot(h, Wh, preferred_element_type=jnp.float32)
        k = 1.0 - draw_ref[i * T_BLK + t, :].astype(jnp.float32)[:, None]
        gh = m * k
        h_m = h * k
        gi_t = gi_ref[t]
        r = jax.nn.sigmoid(gi_t[:, :H] + gh[:, :H])
        z = jax.nn.sigmoid(gi_t[:, H:2 * H] + gh[:, H:2 * H])
        n = jnp.tanh(gi_t[:, 2 * H:] + r * (gh[:, 2 * H:] + bhn))
        h_new = n + z * (h_m - n)
        hs_ref[t] = h_new
        return h_new

    h_final = lax.fori_loop(0, T_BLK, step, h_ref[...], unroll=16)
    h_ref[...] = h_final

    hs = hs_ref[...].reshape(T_BLK * B2, H)
    o = jnp.dot(hs, Wout_ref[...], preferred_element_type=jnp.float32)
    o = (o + bout_ref[0]).reshape(T_BLK, B2, A)
    out_ref[pl.ds(i * T_BLK, T_BLK)] = o

    @pl.when(i == N_BLK - 1)
    def _():
        db = draw_ref[:, B:]
        tv = lax.broadcasted_iota(jnp.int32, (HALF, B), 0)
        first = jnp.min(jnp.where(db > 0, tv, HALF), axis=0)
        L = jnp.max(first)

        Wi = Wi_ref[...]
        Wout = Wout_ref[...]
        bout = bout_ref[0]
        bfold = bfold_ref[0]

        def fstep(t, hf):
            m = jnp.dot(hf, Wh, preferred_element_type=jnp.float32)
            gx = jnp.dot(xB_ref[t], Wi,
                         preferred_element_type=jnp.float32) + bfold
            k = 1.0 - draw_ref[t, B:].astype(jnp.float32)[:, None]
            gh = m * k
            h_m = hf * k
            r = jax.nn.sigmoid(gx[:, :H] + gh[:, :H])
            z = jax.nn.sigmoid(gx[:, H:2 * H] + gh[:, H:2 * H])
            n = jnp.tanh(gx[:, 2 * H:] + r * (gh[:, 2 * H:] + bhn))
            h_new = n + z * (h_m - n)
            hfix_ref[t] = h_new
            return h_new

        def f8(c, hf):
            base = c * 8
            for j in range(8):
                hf = fstep(base + j, hf)
            return hf

        lax.fori_loop(0, (L + 7) // 8, f8, h_ref[:B, :])

        of = jnp.dot(hfix_ref[...].reshape(HALF * B, H), Wout,
                     preferred_element_type=jnp.float32) + bout
        of = of.reshape(HALF, B, A)
        sel = (lax.broadcasted_iota(jnp.int32, (HALF, B, A), 0) < L)
        out_ref[:, B:] = jnp.where(sel, of, out_ref[:, B:])

        keep = (1 - ks_ref[...] * ksn_ref[...]).astype(jnp.float32)
        out_ref[...] = out_ref[...] * keep[:, :, None]


def _fold(a):
    return jnp.swapaxes(a.reshape((2, HALF) + a.shape[1:]), 0, 1).reshape(
        (HALF, 2 * a.shape[1]) + a.shape[2:])


@jax.jit
def kernel(obs, hidden_states, dones, Wi, Wh, bi, bh, Wout, bout):
    x2 = obs.reshape(S, B, D)
    d2 = dones.reshape(S, B)
    split = d2.at[0, :].set(1)
    split_next = jnp.concatenate(
        [split[1:], jnp.ones((1, B), dtype=split.dtype)], axis=0)
    bfold = bi + jnp.concatenate([bh[:2 * H], jnp.zeros((H,), bh.dtype)])
    bhn = bh[2 * H:]

    full = lambda s: pl.BlockSpec(s, lambda i: tuple(0 for _ in s))
    out = pl.pallas_call(
        _gru_kernel,
        grid=(N_BLK,),
        in_specs=[
            pl.BlockSpec((T_BLK, B2, D), lambda i: (i, 0, 0)),
            full((HALF, B2)),
            full((HALF, B2)),
            full((HALF, B2)),
            full((HALF, B, D)),
            full((1, B, H)),
            full((D, 3 * H)),
            full((H, 3 * H)),
            full((1, 3 * H)),
            full((1, H)),
            full((H, A)),
            full((1, A)),
        ],
        out_specs=pl.BlockSpec((HALF, B2, A), lambda i: (0, 0, 0)),
        out_shape=jax.ShapeDtypeStruct((HALF, B2, A), jnp.float32),
        scratch_shapes=[
            pltpu.VMEM((B2, H), jnp.float32),
            pltpu.VMEM((T_BLK, B2, 3 * H), jnp.float32),
            pltpu.VMEM((T_BLK, B2, H), jnp.float32),
            pltpu.VMEM((HALF, B, H), jnp.float32),
        ],
    )(_fold(x2), _fold(d2), _fold(split), _fold(split_next),
      x2[HALF:], hidden_states,
      Wi, Wh, bfold.reshape(1, 3 * H), bhn.reshape(1, H),
      Wout, bout.reshape(1, A))
    return jnp.swapaxes(out.reshape(HALF, 2, B, A), 0, 1).reshape(S * B, A)
